# baseline (speedup 1.0000x reference)
import math

import jax
import jax.numpy as jnp
import numpy as np
from jax.experimental import pallas as pl
from jax.experimental.pallas import tpu as pltpu
from jax.sharding import Mesh, PartitionSpec as P

_BUDGET = 1.0 - 0.05

_N_BISECT = 8
_CHUNK = 1024


def _chunks(dim):
    return [slice(i * _CHUNK, (i + 1) * _CHUNK) for i in range(dim // _CHUNK)]


def _sparsegen_block(x_ref, o_ref, w_ref):
    dim = x_ref.shape[1]
    sls = _chunks(dim)

    acc = x_ref[:, sls[0]]
    for sl in sls[1:]:
        acc = jnp.maximum(acc, x_ref[:, sl])
    rowmax = jnp.max(acc, axis=1, keepdims=True)

    for sl in sls:
        w_ref[:, sl] = (x_ref[:, sl] - rowmax).astype(jnp.bfloat16)

    lo = jnp.full(rowmax.shape, -_BUDGET, jnp.float32)
    hi = jnp.zeros(rowmax.shape, jnp.float32)
    for _ in range(_N_BISECT):
        mid = 0.5 * (lo + hi)
        midh = mid.astype(jnp.bfloat16)
        acc = jnp.maximum(w_ref[:, sls[0]] - midh, jnp.bfloat16(0))
        for sl in sls[1:]:
            acc = acc + jnp.maximum(w_ref[:, sl] - midh, jnp.bfloat16(0))
        r = jnp.sum(acc, axis=1, keepdims=True).astype(jnp.float32)
        pred = r > _BUDGET
        lo = jnp.where(pred, mid, lo)
        hi = jnp.where(pred, hi, mid)

    t = rowmax + 0.5 * (lo + hi)
    c0 = x_ref[:, sls[0]]
    acc = jnp.maximum(c0 - t, 0.0)
    cacc = jnp.where(c0 > t, 1.0, 0.0)
    for sl in sls[1:]:
        c = x_ref[:, sl]
        acc = acc + jnp.maximum(c - t, 0.0)
        cacc = cacc + jnp.where(c > t, 1.0, 0.0)
    r = jnp.sum(acc, axis=1, keepdims=True)
    cnt = jnp.sum(cacc, axis=1, keepdims=True)
    tau = t + (r - _BUDGET) / jnp.maximum(cnt, 1.0)

    for sl in sls:
        o_ref[:, sl] = jnp.maximum(x_ref[:, sl] - tau, 0.0) * (1.0 / _BUDGET)


def _run_rows(x):
    bs, dim = x.shape
    rows_per_block = 32
    return pl.pallas_call(
        _sparsegen_block,
        grid=(bs // rows_per_block,),
        in_specs=[pl.BlockSpec((rows_per_block, dim), lambda i: (i, 0))],
        out_specs=pl.BlockSpec((rows_per_block, dim), lambda i: (i, 0)),
        out_shape=jax.ShapeDtypeStruct((bs, dim), jnp.float32),
        scratch_shapes=[pltpu.VMEM((rows_per_block, dim), jnp.bfloat16)],
    )(x)


@jax.jit
def kernel(input):
    ndev = math.gcd(jax.device_count(), 4)
    mesh = Mesh(np.array(jax.devices()[:ndev]), ("b",))
    f = jax.shard_map(_run_rows, mesh=mesh, in_specs=P("b", None),
                      out_specs=P("b", None), check_vma=False)
    return f(input.astype(jnp.float32))


# speedup vs baseline: 22.6425x; 22.6425x over previous
import jax
import jax.numpy as jnp
from jax.experimental import pallas as pl
from jax.experimental.pallas import tpu as pltpu

_BUDGET = 1.0 - 0.05

_N_BISECT = 8
_CHUNK = 1024


def _chunks(dim):
    return [slice(i * _CHUNK, (i + 1) * _CHUNK) for i in range(dim // _CHUNK)]


def _sparsegen_block(x_ref, o_ref, w_ref):
    dim = x_ref.shape[1]
    sls = _chunks(dim)

    acc = x_ref[:, sls[0]]
    for sl in sls[1:]:
        acc = jnp.maximum(acc, x_ref[:, sl])
    rowmax = jnp.max(acc, axis=1, keepdims=True)

    for sl in sls:
        w_ref[:, sl] = (x_ref[:, sl] - rowmax).astype(jnp.bfloat16)

    lo = jnp.full(rowmax.shape, -_BUDGET, jnp.float32)
    hi = jnp.zeros(rowmax.shape, jnp.float32)
    for _ in range(_N_BISECT):
        mid = 0.5 * (lo + hi)
        midh = mid.astype(jnp.bfloat16)
        acc = jnp.maximum(w_ref[:, sls[0]] - midh, jnp.bfloat16(0))
        for sl in sls[1:]:
            acc = acc + jnp.maximum(w_ref[:, sl] - midh, jnp.bfloat16(0))
        r = jnp.sum(acc, axis=1, keepdims=True).astype(jnp.float32)
        pred = r > _BUDGET
        lo = jnp.where(pred, mid, lo)
        hi = jnp.where(pred, hi, mid)

    t = rowmax + 0.5 * (lo + hi)
    c0 = x_ref[:, sls[0]]
    acc = jnp.maximum(c0 - t, 0.0)
    cacc = jnp.where(c0 > t, 1.0, 0.0)
    for sl in sls[1:]:
        c = x_ref[:, sl]
        acc = acc + jnp.maximum(c - t, 0.0)
        cacc = cacc + jnp.where(c > t, 1.0, 0.0)
    r = jnp.sum(acc, axis=1, keepdims=True)
    cnt = jnp.sum(cacc, axis=1, keepdims=True)
    tau = t + (r - _BUDGET) / jnp.maximum(cnt, 1.0)

    for sl in sls:
        o_ref[:, sl] = jnp.maximum(x_ref[:, sl] - tau, 0.0) * (1.0 / _BUDGET)


def _run_rows(x):
    bs, dim = x.shape
    rows_per_block = 32
    return pl.pallas_call(
        _sparsegen_block,
        grid=(bs // rows_per_block,),
        in_specs=[pl.BlockSpec((rows_per_block, dim), lambda i: (i, 0))],
        out_specs=pl.BlockSpec((rows_per_block, dim), lambda i: (i, 0)),
        out_shape=jax.ShapeDtypeStruct((bs, dim), jnp.float32),
        scratch_shapes=[pltpu.VMEM((rows_per_block, dim), jnp.bfloat16)],
    )(x)


@jax.jit
def kernel(input):
    return _run_rows(input.astype(jnp.float32))
